# baseline (speedup 1.0000x reference)
import jax
import jax.numpy as jnp
from jax import lax
from jax.experimental import pallas as pl
from jax.experimental.pallas import tpu as pltpu

N = 2048
XD = 128
HD = 32
GD = 32
YREP = HD + GD + 1
BLK = 512
GRID = N // BLK

_DN = (((0,), (0,)), ((), ()))
_F32 = jnp.float32


def _body(a_hbm, x_ref, t_ref, w1_ref, b1_ref, wg_ref, bg_ref,
          w00_ref, b00_ref, w10_ref, b10_ref, w01_ref, b01_ref,
          w11_ref, b11_ref,
          out_ref,
          a_s, sems):
    copies = [
        pltpu.make_async_copy(
            a_hbm.at[pl.ds(j * BLK, BLK), :], a_s.at[j], sems.at[j])
        for j in range(GRID)
    ]
    for c in copies:
        c.start()

    t_col = t_ref[...]
    phi = jax.nn.relu(
        jnp.dot(x_ref[...], w1_ref[...], preferred_element_type=_F32)
        + b1_ref[...])
    xl = jnp.dot(t_col * phi, wg_ref[...],
                 preferred_element_type=_F32)

    stats = jnp.zeros((N, 2), _F32)
    for j in range(GRID):
        copies[j].wait()
        to_blk = jnp.concatenate(
            [t_col[j * BLK:(j + 1) * BLK, :],
             jnp.ones((BLK, 1), _F32)], axis=1)
        stats = stats + lax.dot_general(
            a_s[j], to_blk, _DN, preferred_element_type=_F32)

    dinv = lax.rsqrt(stats[:, 1:2] + 1.0)
    z = stats[:, 0:1] / stats[:, 1:2]
    bm = dinv * xl
    cagg = jnp.zeros((N, GD), _F32)
    for j in range(GRID):
        cagg = cagg + lax.dot_general(
            a_s[j], bm[j * BLK:(j + 1) * BLK, :], _DN,
            preferred_element_type=_F32)
    agg = dinv * (cagg + dinv * xl)
    rep_gnn = jax.nn.relu(agg + bg_ref[...])
    rep = jnp.concatenate([phi, rep_gnn, z], axis=1)
    y00 = jax.nn.relu(
        jnp.dot(rep, w00_ref[...], preferred_element_type=_F32)
        + b00_ref[...])
    y10 = jax.nn.relu(
        jnp.dot(rep, w10_ref[...], preferred_element_type=_F32)
        + b10_ref[...])
    y0c = jnp.dot(y00, w01_ref[...],
                  preferred_element_type=_F32) + b01_ref[...]
    y1c = jnp.dot(y10, w11_ref[...],
                  preferred_element_type=_F32) + b11_ref[...]
    out_ref[...] = jnp.concatenate([rep, y0c, y1c], axis=1)


def kernel(X, A, T, W1, b1, Wg, bg, W00, b00, W10, b10, W01, b01, W11, b11):
    t_col = T.reshape(N, 1).astype(_F32)
    full = lambda a: pl.BlockSpec(a.shape, lambda: (0,) * a.ndim)

    vmem_args = (X, t_col, W1, b1.reshape(1, HD), Wg,
                 bg.reshape(1, GD), W00, b00.reshape(1, YREP),
                 W10, b10.reshape(1, YREP), W01, b01.reshape(1, 1),
                 W11, b11.reshape(1, 1))

    out = pl.pallas_call(
        _body,
        in_specs=[pl.BlockSpec(memory_space=pl.ANY)]
        + [full(a) for a in vmem_args],
        out_specs=pl.BlockSpec((N, YREP + 2), lambda: (0, 0)),
        out_shape=jax.ShapeDtypeStruct((N, YREP + 2), _F32),
        scratch_shapes=[pltpu.VMEM((GRID, BLK, N), _F32),
                        pltpu.SemaphoreType.DMA((GRID,))],
    )(A, *vmem_args)

    return (out[:, YREP], out[:, YREP + 1], out[:, :YREP])


# speedup vs baseline: 1.0047x; 1.0047x over previous
import jax
import jax.numpy as jnp
from jax import lax
from jax.experimental import pallas as pl
from jax.experimental.pallas import tpu as pltpu

N = 2048
XD = 128
HD = 32
GD = 32
YREP = HD + GD + 1
BLK = 256
GRID = N // BLK

_DN = (((0,), (0,)), ((), ()))
_F32 = jnp.float32


def _body(a_hbm, x_ref, t_ref, w1_ref, b1_ref, wg_ref, bg_ref,
          w00_ref, b00_ref, w10_ref, b10_ref, w01_ref, b01_ref,
          w11_ref, b11_ref,
          out_ref,
          a_s, sems):
    copies = [
        pltpu.make_async_copy(
            a_hbm.at[pl.ds(j * BLK, BLK), :], a_s.at[j], sems.at[j])
        for j in range(GRID)
    ]
    for c in copies:
        c.start()

    t_col = t_ref[...]
    phi = jax.nn.relu(
        jnp.dot(x_ref[...], w1_ref[...], preferred_element_type=_F32)
        + b1_ref[...])
    xl = jnp.dot(t_col * phi, wg_ref[...],
                 preferred_element_type=_F32)

    stats = jnp.zeros((N, 2), _F32)
    for j in range(GRID):
        copies[j].wait()
        to_blk = jnp.concatenate(
            [t_col[j * BLK:(j + 1) * BLK, :],
             jnp.ones((BLK, 1), _F32)], axis=1)
        stats = stats + lax.dot_general(
            a_s[j], to_blk, _DN, preferred_element_type=_F32)

    dinv = lax.rsqrt(stats[:, 1:2] + 1.0)
    z = stats[:, 0:1] / stats[:, 1:2]
    bm = dinv * xl
    cagg = jnp.zeros((N, GD), _F32)
    for j in range(GRID):
        cagg = cagg + lax.dot_general(
            a_s[j], bm[j * BLK:(j + 1) * BLK, :], _DN,
            preferred_element_type=_F32)
    agg = dinv * (cagg + dinv * xl)
    rep_gnn = jax.nn.relu(agg + bg_ref[...])
    rep = jnp.concatenate([phi, rep_gnn, z], axis=1)
    y00 = jax.nn.relu(
        jnp.dot(rep, w00_ref[...], preferred_element_type=_F32)
        + b00_ref[...])
    y10 = jax.nn.relu(
        jnp.dot(rep, w10_ref[...], preferred_element_type=_F32)
        + b10_ref[...])
    y0c = jnp.dot(y00, w01_ref[...],
                  preferred_element_type=_F32) + b01_ref[...]
    y1c = jnp.dot(y10, w11_ref[...],
                  preferred_element_type=_F32) + b11_ref[...]
    out_ref[...] = jnp.concatenate([rep, y0c, y1c], axis=1)


def kernel(X, A, T, W1, b1, Wg, bg, W00, b00, W10, b10, W01, b01, W11, b11):
    t_col = T.reshape(N, 1).astype(_F32)
    full = lambda a: pl.BlockSpec(a.shape, lambda: (0,) * a.ndim)

    vmem_args = (X, t_col, W1, b1.reshape(1, HD), Wg,
                 bg.reshape(1, GD), W00, b00.reshape(1, YREP),
                 W10, b10.reshape(1, YREP), W01, b01.reshape(1, 1),
                 W11, b11.reshape(1, 1))

    out = pl.pallas_call(
        _body,
        in_specs=[pl.BlockSpec(memory_space=pl.ANY)]
        + [full(a) for a in vmem_args],
        out_specs=pl.BlockSpec((N, YREP + 2), lambda: (0, 0)),
        out_shape=jax.ShapeDtypeStruct((N, YREP + 2), _F32),
        scratch_shapes=[pltpu.VMEM((GRID, BLK, N), _F32),
                        pltpu.SemaphoreType.DMA((GRID,))],
    )(A, *vmem_args)

    return (out[:, YREP], out[:, YREP + 1], out[:, :YREP])
